# baseline (speedup 1.0000x reference)
import jax
import jax.numpy as jnp
from jax import lax
from jax.experimental import pallas as pl
from jax.experimental.pallas import tpu as pltpu

B = 64
B2 = 2 * B
EDD = 2048
EDS = 1024
HALF = EDS // 2
BN_EPS = 1e-5
COS_EPS = 1e-8

_DN_T = (((1,), (1,)), ((), ()))
_DN = (((1,), (0,)), ((), ()))


def _fused_kernel(x_ref, y_ref, w1_ref, w_hbm, out_ref, xy, w2, sem):
    c2 = pltpu.make_async_copy(w_hbm.at[pl.ds(HALF, HALF), :], w2, sem)
    c2.start()

    xy[0:B, :] = x_ref[...]
    xy[B:B2, :] = y_ref[...]

    row = lax.broadcasted_iota(jnp.int32, (B2, HALF), 0)
    is_x = row < B
    sel_i = lax.broadcasted_iota(jnp.int32, (2, B2), 0)
    sel_j = lax.broadcasted_iota(jnp.int32, (2, B2), 1)
    sel = jnp.where((sel_j // B) == sel_i, 1.0, 0.0).astype(jnp.float32)

    ones_col = jnp.ones((HALF, 1), dtype=jnp.float32)
    lane = lax.broadcasted_iota(jnp.int32, (B2, HALF), 1)
    at_block_start = (lane % 4) == 0
    low = jnp.full((B2, HALF), -2.0, dtype=jnp.float32)

    def masked(hh):
        s1 = lax.dot_general(sel, hh, _DN,
                             preferred_element_type=jnp.float32)
        s2 = lax.dot_general(sel, hh * hh, _DN,
                             preferred_element_type=jnp.float32)
        mu2 = s1 * (1.0 / B)
        var2 = s2 * (1.0 / B) - mu2 * mu2
        scale2 = lax.rsqrt(var2 + BN_EPS)
        mu = jnp.where(is_x, mu2[0:1, :], mu2[1:2, :])
        scale = jnp.where(is_x, scale2[0:1, :], scale2[1:2, :])
        th = jnp.tanh((hh - mu) * scale)
        a = jnp.maximum(th, pltpu.roll(th, HALF - 1, 1))
        bm = jnp.maximum(a, pltpu.roll(a, HALF - 2, 1))
        c = jnp.where(at_block_start, bm, low)
        c = jnp.maximum(c, pltpu.roll(c, 1, 1))
        bmax = jnp.maximum(c, pltpu.roll(c, 2, 1))
        return jnp.where(th == bmax, th, 0.0)

    def partials(m):
        p = m * pltpu.roll(m, B, 0)
        n = m * m
        P = lax.dot_general(p, ones_col, _DN,
                            preferred_element_type=jnp.float32)
        N = lax.dot_general(n, ones_col, _DN,
                            preferred_element_type=jnp.float32)
        return P, N

    hh1 = lax.dot_general(xy[...], w1_ref[...], _DN_T,
                          preferred_element_type=jnp.float32)
    P1, N1 = partials(masked(hh1))

    c2.wait()
    hh2 = lax.dot_general(xy[...], w2[...], _DN_T,
                          preferred_element_type=jnp.float32)
    P2, N2 = partials(masked(hh2))

    P = P1 + P2
    N = N1 + N2
    dot = P[0:B, :]
    nxc = jnp.maximum(jnp.sqrt(N[0:B, :]), COS_EPS)
    nyc = jnp.maximum(jnp.sqrt(N[B:B2, :]), COS_EPS)
    out_ref[...] = (dot / (nxc * nyc)).reshape(B)


def kernel(x, y, W, b, gamma_x, beta_x, gamma_y, beta_y):
    out = pl.pallas_call(
        _fused_kernel,
        grid=(1,),
        in_specs=[
            pl.BlockSpec((B, EDD), lambda i: (0, 0)),
            pl.BlockSpec((B, EDD), lambda i: (0, 0)),
            pl.BlockSpec((HALF, EDD), lambda i: (0, 0)),
            pl.BlockSpec(memory_space=pltpu.MemorySpace.HBM),
        ],
        out_specs=pl.BlockSpec((B,), lambda i: (0,)),
        out_shape=jax.ShapeDtypeStruct((B,), jnp.float32),
        scratch_shapes=[
            pltpu.VMEM((B2, EDD), jnp.float32),
            pltpu.VMEM((HALF, EDD), jnp.float32),
            pltpu.SemaphoreType.DMA,
        ],
    )(x, y, W, W)
    return out


# speedup vs baseline: 1.0039x; 1.0039x over previous
import jax
import jax.numpy as jnp
from jax import lax
from jax.experimental import pallas as pl
from jax.experimental.pallas import tpu as pltpu

B = 64
B2 = 2 * B
EDD = 2048
EDS = 1024
BN_EPS = 1e-5
COS_EPS = 1e-8

_DN_T = (((1,), (1,)), ((), ()))
_DN = (((1,), (0,)), ((), ()))


def _fused_kernel(x_ref, y_ref, w_ref, out_ref, xy):
    xy[0:B, :] = x_ref[...]
    xy[B:B2, :] = y_ref[...]

    row = lax.broadcasted_iota(jnp.int32, (B2, EDS), 0)
    is_x = row < B
    sel_i = lax.broadcasted_iota(jnp.int32, (2, B2), 0)
    sel_j = lax.broadcasted_iota(jnp.int32, (2, B2), 1)
    sel = jnp.where((sel_j // B) == sel_i, 1.0, 0.0).astype(jnp.float32)

    ones_col = jnp.ones((EDS, 1), dtype=jnp.float32)
    lane = lax.broadcasted_iota(jnp.int32, (B2, EDS), 1)
    at_block_start = (lane % 4) == 0
    low = jnp.full((B2, EDS), -2.0, dtype=jnp.float32)

    w = w_ref[...]
    hh = lax.dot_general(xy[...], w, _DN_T,
                         preferred_element_type=jnp.float32)

    s1 = lax.dot_general(sel, hh, _DN,
                         preferred_element_type=jnp.float32)
    s2 = lax.dot_general(sel, hh * hh, _DN,
                         preferred_element_type=jnp.float32)
    mu2 = s1 * (1.0 / B)
    var2 = s2 * (1.0 / B) - mu2 * mu2
    scale2 = lax.rsqrt(var2 + BN_EPS)
    mu = jnp.where(is_x, mu2[0:1, :], mu2[1:2, :])
    scale = jnp.where(is_x, scale2[0:1, :], scale2[1:2, :])
    th = jnp.tanh((hh - mu) * scale)

    a = jnp.maximum(th, pltpu.roll(th, EDS - 1, 1))
    bm = jnp.maximum(a, pltpu.roll(a, EDS - 2, 1))
    c = jnp.where(at_block_start, bm, low)
    c = jnp.maximum(c, pltpu.roll(c, 1, 1))
    bmax = jnp.maximum(c, pltpu.roll(c, 2, 1))
    m = jnp.where(th == bmax, th, 0.0)

    p = m * pltpu.roll(m, B, 0)
    n = m * m
    P = lax.dot_general(p, ones_col, _DN,
                        preferred_element_type=jnp.float32)
    N = lax.dot_general(n, ones_col, _DN,
                        preferred_element_type=jnp.float32)
    dot = P[0:B, :]
    nxc = jnp.maximum(jnp.sqrt(N[0:B, :]), COS_EPS)
    nyc = jnp.maximum(jnp.sqrt(N[B:B2, :]), COS_EPS)
    out_ref[...] = (dot / (nxc * nyc)).reshape(B)


def kernel(x, y, W, b, gamma_x, beta_x, gamma_y, beta_y):
    out = pl.pallas_call(
        _fused_kernel,
        in_specs=[
            pl.BlockSpec((B, EDD), lambda: (0, 0)),
            pl.BlockSpec((B, EDD), lambda: (0, 0)),
            pl.BlockSpec((EDS, EDD), lambda: (0, 0)),
        ],
        out_specs=pl.BlockSpec((B,), lambda: (0,)),
        out_shape=jax.ShapeDtypeStruct((B,), jnp.float32),
        scratch_shapes=[
            pltpu.VMEM((B2, EDD), jnp.float32),
        ],
    )(x, y, W)
    return out
